# Initial kernel scaffold; baseline (speedup 1.0000x reference)
import jax
import jax.numpy as jnp
from jax.experimental import pallas as pl


def kernel(x, edge_index, params):
    raise NotImplementedError("write your pallas kernel here")


# speedup vs baseline: 2.8572x; 2.8572x over previous
import functools

import jax
import jax.numpy as jnp
from jax import lax
from jax.experimental import pallas as pl
from jax.experimental.pallas import tpu as pltpu
from jax.experimental.pallas import tpu_sc as plsc

N = 50000
E = 800000
NP = 50176
RB = 512
NB = NP // RB
NWORK = 32
BATCH = 128
EPW = 25088
NB_E = EPW // BATCH
EP = NWORK * EPW
RPT = NP // 16

LAYER_DIMS = [(4, 16),
              (16, 16), (16, 16), (16, 16), (16, 16),
              (16, 32), (32, 32), (32, 32), (32, 32), (32, 32),
              (32, 64), (64, 64), (64, 64), (64, 64), (64, 64),
              (64, 128), (128, 128), (128, 128), (128, 128), (128, 128)]
SKIP_LAYERS = frozenset({2, 4, 7, 9, 12, 14, 17, 19})
RES_START = frozenset({1, 3, 6, 8, 11, 13, 16, 18})


def _chunks(cout):
    wc = min(cout, 16)
    return cout // wc, wc


@functools.lru_cache(maxsize=None)
def _dense_first(cin, cout):
    k_, wc = _chunks(cout)

    def body(h_ref, w_ref, g_ref):
        g = jnp.dot(h_ref[...], w_ref[0], preferred_element_type=jnp.float32)
        g_ref[...] = g[None]

    return pl.pallas_call(
        body,
        grid=(NB, k_),
        in_specs=[
            pl.BlockSpec((RB, cin), lambda i, k: (i, 0)),
            pl.BlockSpec((1, cin, wc), lambda i, k: (k, 0, 0)),
        ],
        out_specs=pl.BlockSpec((1, RB, wc), lambda i, k: (k, i, 0)),
        out_shape=jax.ShapeDtypeStruct((k_, NP, wc), jnp.float32),
    )


@functools.lru_cache(maxsize=None)
def _combine(cin, cout, has_skip, next_cout):
    k_, wc = _chunks(cout)
    if next_cout is not None:
        kn, wcn = _chunks(next_cout)
    else:
        kn, wcn = 1, 0

    def body(*refs):
        it = iter(refs)
        h_ref = next(it)
        p_ref = next(it)
        ws_ref = next(it)
        gb_ref = next(it)
        skip_ref = next(it) if has_skip else None
        wn_ref = next(it) if next_cout is not None else None
        hout_ref = next(it)
        gout_ref = next(it) if next_cout is not None else None

        p = p_ref[...]
        neigh = jnp.concatenate([p[c, 0] + p[c, 1] for c in range(k_)], axis=-1)
        out = jnp.dot(h_ref[...], ws_ref[...], preferred_element_type=jnp.float32)
        out = (out + neigh) * gb_ref[0] + gb_ref[1]
        if skip_ref is not None:
            out = out + skip_ref[...]
        out = jnp.maximum(out, 0.0)
        rows = pl.program_id(0) * RB + lax.broadcasted_iota(jnp.int32, (RB, 1), 0)
        out = jnp.where(rows < N, out, 0.0)
        hout_ref[...] = out
        if gout_ref is not None:
            g = jnp.dot(out, wn_ref[0], preferred_element_type=jnp.float32)
            gout_ref[...] = g[None]

    in_specs = [
        pl.BlockSpec((RB, cin), lambda i, k: (i, 0)),
        pl.BlockSpec((k_, 2, RB, wc), lambda i, k: (0, 0, i, 0)),
        pl.BlockSpec((cin, cout), lambda i, k: (0, 0)),
        pl.BlockSpec((2, cout), lambda i, k: (0, 0)),
    ]
    if has_skip:
        in_specs.append(pl.BlockSpec((RB, cout), lambda i, k: (i, 0)))
    out_specs = [pl.BlockSpec((RB, cout), lambda i, k: (i, 0))]
    out_shape = [jax.ShapeDtypeStruct((NP, cout), jnp.float32)]
    if next_cout is not None:
        in_specs.append(pl.BlockSpec((1, cout, wcn), lambda i, k: (k, 0, 0)))
        out_specs.append(pl.BlockSpec((1, RB, wcn), lambda i, k: (k, i, 0)))
        out_shape.append(jax.ShapeDtypeStruct((kn, NP, wcn), jnp.float32))

    return pl.pallas_call(
        body,
        grid=(NB, kn),
        in_specs=in_specs,
        out_specs=out_specs,
        out_shape=out_shape,
    )


@functools.lru_cache(maxsize=None)
def _sc_scatter(k_, wc):
    mesh = plsc.VectorSubcoreMesh(core_axis_name="c", subcore_axis_name="s")

    @functools.partial(
        pl.kernel,
        mesh=mesh,
        out_type=jax.ShapeDtypeStruct((k_, 2, NP, wc), jnp.float32),
        scratch_types=[
            pltpu.VMEM((NB_E, BATCH), jnp.int32),
            pltpu.VMEM((NB_E, BATCH), jnp.int32),
            pltpu.VMEM((BATCH, wc), jnp.float32),
            pltpu.VMEM((BATCH, wc), jnp.float32),
            pltpu.VMEM_SHARED((NP, wc), jnp.float32),
            pltpu.SemaphoreType.DMA,
        ],
        compiler_params=pltpu.CompilerParams(use_tc_tiling_on_sc=False),
    )
    def sc_fn(g_hbm, src_hbm, dst_hbm, zer_hbm, out_hbm,
              src_v, dst_v, rows_v, zbuf_v, acc, sem):
        c = lax.axis_index("c")
        s = lax.axis_index("s")
        wid = c * 16 + s
        pltpu.sync_copy(src_hbm.at[wid], src_v)
        pltpu.sync_copy(dst_hbm.at[wid], dst_v)
        pltpu.sync_copy(zer_hbm, zbuf_v)
        base = s * RPT
        for k in range(k_):
            def zbody(i, _):
                pltpu.sync_copy(zbuf_v, acc.at[pl.ds(base + i * BATCH, BATCH)])
                return 0
            lax.fori_loop(0, RPT // BATCH, zbody, 0)
            rem = RPT % BATCH
            if rem:
                pltpu.sync_copy(zbuf_v.at[pl.ds(0, rem)],
                                acc.at[pl.ds(base + RPT - rem, rem)])
            plsc.subcore_barrier()

            def sbody(j, _):
                pltpu.async_copy(g_hbm.at[k].at[src_v.at[j]], rows_v, sem).wait()
                pltpu.sync_copy(rows_v, acc.at[dst_v.at[j]], add=True)
                return 0
            lax.fori_loop(0, NB_E, sbody, 0)
            plsc.subcore_barrier()

            pltpu.sync_copy(acc.at[pl.ds(base, RPT)],
                            out_hbm.at[k].at[c].at[pl.ds(base, RPT)])
            plsc.subcore_barrier()

    return sc_fn


def _chunk_w(w):
    cin, cout = w.shape
    k_, wc = _chunks(cout)
    return w.reshape(cin, k_, wc).transpose(1, 0, 2)


def kernel(x, edge_index, params):
    src = edge_index[0]
    dst = edge_index[1]
    pad = jnp.full((EP - E,), N, dtype=jnp.int32)
    src_p = jnp.concatenate([src, pad]).reshape(NWORK, NB_E, BATCH)
    dst_p = jnp.concatenate([dst, pad]).reshape(NWORK, NB_E, BATCH)
    zeros16 = jnp.zeros((BATCH, 16), jnp.float32)

    h = jnp.pad(x, ((0, NP - N), (0, 0)))
    g = _dense_first(4, 16)(h, _chunk_w(params[0][1]))

    res_in = None
    for L, (cin, cout) in enumerate(LAYER_DIMS):
        w_self, w_neigh, gamma, beta = params[L]
        k_, wc = _chunks(cout)
        partials = _sc_scatter(k_, wc)(g, src_p, dst_p, zeros16)

        if L in RES_START:
            res_in = h
        gb = jnp.stack([gamma, beta])
        has_skip = L in SKIP_LAYERS
        next_cout = LAYER_DIMS[L + 1][1] if L + 1 < len(LAYER_DIMS) else None
        args = [h, partials, w_self, gb]
        if has_skip:
            args.append(res_in)
        if next_cout is not None:
            args.append(_chunk_w(params[L + 1][1]))
            h, g = _combine(cin, cout, has_skip, next_cout)(*args)
        else:
            (h,) = _combine(cin, cout, has_skip, None)(*args)

    return h[:N]


# speedup vs baseline: 4.7472x; 1.6615x over previous
import functools

import jax
import jax.numpy as jnp
from jax import lax
from jax.experimental import pallas as pl
from jax.experimental.pallas import tpu as pltpu
from jax.experimental.pallas import tpu_sc as plsc

N = 50000
E = 800000
NP = 50176
RB = 512
NB = NP // RB
NWORK = 32
BATCH = 128
EPW = 25088
NB_E = EPW // BATCH
EP = NWORK * EPW
RPT = NP // 16

LAYER_DIMS = [(4, 16),
              (16, 16), (16, 16), (16, 16), (16, 16),
              (16, 32), (32, 32), (32, 32), (32, 32), (32, 32),
              (32, 64), (64, 64), (64, 64), (64, 64), (64, 64),
              (64, 128), (128, 128), (128, 128), (128, 128), (128, 128)]
SKIP_LAYERS = frozenset({2, 4, 7, 9, 12, 14, 17, 19})
RES_START = frozenset({1, 3, 6, 8, 11, 13, 16, 18})


def _chunks(cout):
    wc = min(cout, 16)
    return cout // wc, wc


@functools.lru_cache(maxsize=None)
def _dense_first(cin, cout):
    k_, wc = _chunks(cout)

    def body(h_ref, w_ref, g_ref):
        g = jnp.dot(h_ref[...], w_ref[0], preferred_element_type=jnp.float32)
        g_ref[...] = g[None]

    return pl.pallas_call(
        body,
        grid=(NB, k_),
        in_specs=[
            pl.BlockSpec((RB, cin), lambda i, k: (i, 0)),
            pl.BlockSpec((1, cin, wc), lambda i, k: (k, 0, 0)),
        ],
        out_specs=pl.BlockSpec((1, RB, wc), lambda i, k: (k, i, 0)),
        out_shape=jax.ShapeDtypeStruct((k_, NP, wc), jnp.float32),
    )


@functools.lru_cache(maxsize=None)
def _combine(cin, cout, has_skip, next_cout):
    k_, wc = _chunks(cout)
    if next_cout is not None:
        kn, wcn = _chunks(next_cout)
    else:
        kn, wcn = 1, 0

    def body(*refs):
        it = iter(refs)
        h_ref = next(it)
        p_ref = next(it)
        ws_ref = next(it)
        gb_ref = next(it)
        skip_ref = next(it) if has_skip else None
        wn_ref = next(it) if next_cout is not None else None
        hout_ref = next(it)
        gout_ref = next(it) if next_cout is not None else None

        p = p_ref[...]
        neigh = jnp.concatenate([p[c, 0] + p[c, 1] for c in range(k_)], axis=-1)
        out = jnp.dot(h_ref[...], ws_ref[...], preferred_element_type=jnp.float32)
        out = (out + neigh) * gb_ref[0] + gb_ref[1]
        if skip_ref is not None:
            out = out + skip_ref[...]
        out = jnp.maximum(out, 0.0)
        rows = pl.program_id(0) * RB + lax.broadcasted_iota(jnp.int32, (RB, 1), 0)
        out = jnp.where(rows < N, out, 0.0)
        hout_ref[...] = out
        if gout_ref is not None:
            g = jnp.dot(out, wn_ref[0], preferred_element_type=jnp.float32)
            gout_ref[...] = g[None]

    in_specs = [
        pl.BlockSpec((RB, cin), lambda i, k: (i, 0)),
        pl.BlockSpec((k_, 2, RB, wc), lambda i, k: (0, 0, i, 0)),
        pl.BlockSpec((cin, cout), lambda i, k: (0, 0)),
        pl.BlockSpec((2, cout), lambda i, k: (0, 0)),
    ]
    if has_skip:
        in_specs.append(pl.BlockSpec((RB, cout), lambda i, k: (i, 0)))
    out_specs = [pl.BlockSpec((RB, cout), lambda i, k: (i, 0))]
    out_shape = [jax.ShapeDtypeStruct((NP, cout), jnp.float32)]
    if next_cout is not None:
        in_specs.append(pl.BlockSpec((1, cout, wcn), lambda i, k: (k, 0, 0)))
        out_specs.append(pl.BlockSpec((1, RB, wcn), lambda i, k: (k, i, 0)))
        out_shape.append(jax.ShapeDtypeStruct((kn, NP, wcn), jnp.float32))

    return pl.pallas_call(
        body,
        grid=(NB, kn),
        in_specs=in_specs,
        out_specs=out_specs,
        out_shape=out_shape,
    )


NBUF = 14
ROUNDS = NB_E // NBUF


@functools.lru_cache(maxsize=None)
def _sc_scatter(k_, wc):
    mesh = plsc.VectorSubcoreMesh(core_axis_name="c", subcore_axis_name="s")

    @functools.partial(
        pl.kernel,
        mesh=mesh,
        out_type=jax.ShapeDtypeStruct((k_, 2, NP, wc), jnp.float32),
        scratch_types=[
            pltpu.VMEM((NB_E, BATCH), jnp.int32),
            pltpu.VMEM((NB_E, BATCH), jnp.int32),
            pltpu.VMEM((NBUF, BATCH, wc), jnp.float32),
            pltpu.VMEM((BATCH, wc), jnp.float32),
            pltpu.VMEM_SHARED((NP, wc), jnp.float32),
            [pltpu.SemaphoreType.DMA] * NBUF,
            [pltpu.SemaphoreType.DMA] * NBUF,
            pltpu.SemaphoreType.DMA,
        ],
        compiler_params=pltpu.CompilerParams(use_tc_tiling_on_sc=False),
    )
    def sc_fn(g_hbm, src_hbm, dst_hbm, zer_hbm, out_hbm,
              src_v, dst_v, rows_v, zbuf_v, acc, sem_g, sem_s, sem_z):
        c = lax.axis_index("c")
        s = lax.axis_index("s")
        wid = c * 16 + s
        pltpu.sync_copy(src_hbm.at[wid], src_v)
        pltpu.sync_copy(dst_hbm.at[wid], dst_v)
        pltpu.sync_copy(zer_hbm, zbuf_v)
        base = s * RPT
        nz = RPT // BATCH
        rem = RPT % BATCH

        def zero_stripe():
            zd = []
            for i in range(nz):
                zd.append(pltpu.async_copy(
                    zbuf_v, acc.at[pl.ds(base + i * BATCH, BATCH)], sem_z))
            if rem:
                zd.append(pltpu.async_copy(
                    zbuf_v.at[pl.ds(0, rem)],
                    acc.at[pl.ds(base + RPT - rem, rem)], sem_z))
            for d in zd:
                d.wait()

        def gather(k, b, j):
            return pltpu.async_copy(
                g_hbm.at[k].at[src_v.at[j]], rows_v.at[b], sem_g[b])

        def scat(b, j):
            return pltpu.async_copy(
                rows_v.at[b], acc.at[dst_v.at[j]], sem_s[b], add=True)

        zero_stripe()
        for k in range(k_):
            plsc.subcore_barrier()

            for b in range(NBUF):
                gather(k, b, b)

            def round_body(jb, _):
                for b in range(NBUF):
                    pltpu.make_async_copy(
                        g_hbm.at[k].at[src_v.at[jb * NBUF + b]],
                        rows_v.at[b], sem_g[b]).wait()
                    scat(b, jb * NBUF + b)
                for b in range(NBUF):
                    pltpu.make_async_copy(
                        rows_v.at[b], acc.at[dst_v.at[jb * NBUF + b]],
                        sem_s[b]).wait()
                    gather(k, b, (jb + 1) * NBUF + b)
                return 0

            lax.fori_loop(0, ROUNDS - 1, round_body, 0)
            jl = (ROUNDS - 1) * NBUF
            tail = []
            for b in range(NBUF):
                pltpu.make_async_copy(
                    g_hbm.at[k].at[src_v.at[jl + b]], rows_v.at[b],
                    sem_g[b]).wait()
                tail.append(scat(b, jl + b))
            for d in tail:
                d.wait()

            plsc.subcore_barrier()
            pltpu.sync_copy(acc.at[pl.ds(base, RPT)],
                            out_hbm.at[k].at[c].at[pl.ds(base, RPT)])
            if k + 1 < k_:
                zero_stripe()

    return sc_fn


def _chunk_w(w):
    cin, cout = w.shape
    k_, wc = _chunks(cout)
    return w.reshape(cin, k_, wc).transpose(1, 0, 2)


def kernel(x, edge_index, params):
    src = edge_index[0]
    dst = edge_index[1]
    pad = jnp.full((EP - E,), N, dtype=jnp.int32)
    src_p = jnp.concatenate([src, pad]).reshape(NWORK, NB_E, BATCH)
    dst_p = jnp.concatenate([dst, pad]).reshape(NWORK, NB_E, BATCH)
    zeros16 = jnp.zeros((BATCH, 16), jnp.float32)

    h = jnp.pad(x, ((0, NP - N), (0, 0)))
    g = _dense_first(4, 16)(h, _chunk_w(params[0][1]))

    res_in = None
    for L, (cin, cout) in enumerate(LAYER_DIMS):
        w_self, w_neigh, gamma, beta = params[L]
        k_, wc = _chunks(cout)
        partials = _sc_scatter(k_, wc)(g, src_p, dst_p, zeros16)

        if L in RES_START:
            res_in = h
        gb = jnp.stack([gamma, beta])
        has_skip = L in SKIP_LAYERS
        next_cout = LAYER_DIMS[L + 1][1] if L + 1 < len(LAYER_DIMS) else None
        args = [h, partials, w_self, gb]
        if has_skip:
            args.append(res_in)
        if next_cout is not None:
            args.append(_chunk_w(params[L + 1][1]))
            h, g = _combine(cin, cout, has_skip, next_cout)(*args)
        else:
            (h,) = _combine(cin, cout, has_skip, None)(*args)

    return h[:N]


# speedup vs baseline: 4.7953x; 1.0101x over previous
import functools

import jax
import jax.numpy as jnp
from jax import lax
from jax.experimental import pallas as pl
from jax.experimental.pallas import tpu as pltpu
from jax.experimental.pallas import tpu_sc as plsc

N = 50000
E = 800000
NP = 50176
RB = 512
NB = NP // RB
NWORK = 32
BATCH = 128
EPW = 25088
NB_E = EPW // BATCH
EP = NWORK * EPW
RPT = NP // 16

LAYER_DIMS = [(4, 16),
              (16, 16), (16, 16), (16, 16), (16, 16),
              (16, 32), (32, 32), (32, 32), (32, 32), (32, 32),
              (32, 64), (64, 64), (64, 64), (64, 64), (64, 64),
              (64, 128), (128, 128), (128, 128), (128, 128), (128, 128)]
SKIP_LAYERS = frozenset({2, 4, 7, 9, 12, 14, 17, 19})
RES_START = frozenset({1, 3, 6, 8, 11, 13, 16, 18})


def _chunks(cout):
    wc = min(cout, 16)
    return cout // wc, wc


@functools.lru_cache(maxsize=None)
def _dense_first(cin, cout):
    k_, wc = _chunks(cout)

    def body(h_ref, w_ref, g_ref):
        g = jnp.dot(h_ref[...], w_ref[0], preferred_element_type=jnp.float32)
        g_ref[...] = g[None]

    return pl.pallas_call(
        body,
        grid=(NB, k_),
        in_specs=[
            pl.BlockSpec((RB, cin), lambda i, k: (i, 0)),
            pl.BlockSpec((1, cin, wc), lambda i, k: (k, 0, 0)),
        ],
        out_specs=pl.BlockSpec((1, RB, wc), lambda i, k: (k, i, 0)),
        out_shape=jax.ShapeDtypeStruct((k_, NP, wc), jnp.float32),
    )


@functools.lru_cache(maxsize=None)
def _combine(cin, cout, has_skip, next_cout):
    k_, wc = _chunks(cout)
    if next_cout is not None:
        kn, wcn = _chunks(next_cout)
    else:
        kn, wcn = 1, 0

    def body(*refs):
        it = iter(refs)
        h_ref = next(it)
        p_ref = next(it)
        ws_ref = next(it)
        gb_ref = next(it)
        skip_ref = next(it) if has_skip else None
        wn_ref = next(it) if next_cout is not None else None
        hout_ref = next(it)
        gout_ref = next(it) if next_cout is not None else None

        p = p_ref[...]
        neigh = jnp.concatenate([p[c, 0] + p[c, 1] for c in range(k_)], axis=-1)
        out = jnp.dot(h_ref[...], ws_ref[...], preferred_element_type=jnp.float32)
        out = (out + neigh) * gb_ref[0] + gb_ref[1]
        if skip_ref is not None:
            out = out + skip_ref[...]
        out = jnp.maximum(out, 0.0)
        rows = pl.program_id(0) * RB + lax.broadcasted_iota(jnp.int32, (RB, 1), 0)
        out = jnp.where(rows < N, out, 0.0)
        hout_ref[...] = out
        if gout_ref is not None:
            g = jnp.dot(out, wn_ref[0], preferred_element_type=jnp.float32)
            gout_ref[...] = g[None]

    in_specs = [
        pl.BlockSpec((RB, cin), lambda i, k: (i, 0)),
        pl.BlockSpec((k_, 2, RB, wc), lambda i, k: (0, 0, i, 0)),
        pl.BlockSpec((cin, cout), lambda i, k: (0, 0)),
        pl.BlockSpec((2, cout), lambda i, k: (0, 0)),
    ]
    if has_skip:
        in_specs.append(pl.BlockSpec((RB, cout), lambda i, k: (i, 0)))
    out_specs = [pl.BlockSpec((RB, cout), lambda i, k: (i, 0))]
    out_shape = [jax.ShapeDtypeStruct((NP, cout), jnp.float32)]
    if next_cout is not None:
        in_specs.append(pl.BlockSpec((1, cout, wcn), lambda i, k: (k, 0, 0)))
        out_specs.append(pl.BlockSpec((1, RB, wcn), lambda i, k: (k, i, 0)))
        out_shape.append(jax.ShapeDtypeStruct((kn, NP, wcn), jnp.float32))

    return pl.pallas_call(
        body,
        grid=(NB, kn),
        in_specs=in_specs,
        out_specs=out_specs,
        out_shape=out_shape,
    )


SUB = 2
BOP = SUB * BATCH
NOP = EPW // BOP
NBUF = 7
ROUNDS = NOP // NBUF
ZR = 64


@functools.lru_cache(maxsize=None)
def _sc_scatter(k_, wc):
    mesh = plsc.VectorSubcoreMesh(core_axis_name="c", subcore_axis_name="s")

    @functools.partial(
        pl.kernel,
        mesh=mesh,
        out_type=jax.ShapeDtypeStruct((k_, 2, NP, wc), jnp.float32),
        scratch_types=[
            pltpu.VMEM((EPW,), jnp.int32),
            pltpu.VMEM((EPW,), jnp.int32),
            pltpu.VMEM((NBUF, BOP, wc), jnp.float32),
            pltpu.VMEM((ZR, wc), jnp.float32),
            pltpu.VMEM_SHARED((NP, wc), jnp.float32),
            [pltpu.SemaphoreType.DMA] * NBUF,
            [pltpu.SemaphoreType.DMA] * NBUF,
            pltpu.SemaphoreType.DMA,
        ],
        compiler_params=pltpu.CompilerParams(use_tc_tiling_on_sc=False),
    )
    def sc_fn(g_hbm, src_hbm, dst_hbm, zer_hbm, out_hbm,
              src_v, dst_v, rows_v, zbuf_v, acc, sem_g, sem_s, sem_z):
        c = lax.axis_index("c")
        s = lax.axis_index("s")
        wid = c * 16 + s
        pltpu.sync_copy(src_hbm.at[wid], src_v)
        pltpu.sync_copy(dst_hbm.at[wid], dst_v)
        pltpu.sync_copy(zer_hbm, zbuf_v)
        base = s * RPT
        nz = RPT // ZR
        rem = RPT % ZR

        def zero_stripe():
            zd = []
            for i in range(nz):
                zd.append(pltpu.async_copy(
                    zbuf_v, acc.at[pl.ds(base + i * ZR, ZR)], sem_z))
            if rem:
                zd.append(pltpu.async_copy(
                    zbuf_v.at[pl.ds(0, rem)],
                    acc.at[pl.ds(base + RPT - rem, rem)], sem_z))
            for d in zd:
                d.wait()

        def gather(k, b, j):
            return pltpu.async_copy(
                g_hbm.at[k].at[src_v.at[pl.ds(j * BOP, BOP)]],
                rows_v.at[b], sem_g[b])

        def scat(b, j):
            return pltpu.async_copy(
                rows_v.at[b], acc.at[dst_v.at[pl.ds(j * BOP, BOP)]],
                sem_s[b], add=True)

        zero_stripe()
        for k in range(k_):
            plsc.subcore_barrier()

            for b in range(NBUF):
                gather(k, b, b)

            def round_body(jb, _):
                for b in range(NBUF):
                    pltpu.make_async_copy(
                        g_hbm.at[k].at[src_v.at[pl.ds((jb * NBUF + b) * BOP, BOP)]],
                        rows_v.at[b], sem_g[b]).wait()
                    scat(b, jb * NBUF + b)
                for b in range(NBUF):
                    pltpu.make_async_copy(
                        rows_v.at[b], acc.at[dst_v.at[pl.ds((jb * NBUF + b) * BOP, BOP)]],
                        sem_s[b]).wait()
                    gather(k, b, (jb + 1) * NBUF + b)
                return 0

            lax.fori_loop(0, ROUNDS - 1, round_body, 0)
            jl = (ROUNDS - 1) * NBUF
            tail = []
            for b in range(NBUF):
                pltpu.make_async_copy(
                    g_hbm.at[k].at[src_v.at[pl.ds((jl + b) * BOP, BOP)]],
                    rows_v.at[b], sem_g[b]).wait()
                tail.append(scat(b, jl + b))
            for d in tail:
                d.wait()

            plsc.subcore_barrier()
            pltpu.sync_copy(acc.at[pl.ds(base, RPT)],
                            out_hbm.at[k].at[c].at[pl.ds(base, RPT)])
            if k + 1 < k_:
                zero_stripe()

    return sc_fn


def _chunk_w(w):
    cin, cout = w.shape
    k_, wc = _chunks(cout)
    return w.reshape(cin, k_, wc).transpose(1, 0, 2)


def kernel(x, edge_index, params):
    src = edge_index[0]
    dst = edge_index[1]
    pad = jnp.full((EP - E,), N, dtype=jnp.int32)
    src_p = jnp.concatenate([src, pad]).reshape(NWORK, EPW)
    dst_p = jnp.concatenate([dst, pad]).reshape(NWORK, EPW)
    zeros16 = jnp.zeros((ZR, 16), jnp.float32)

    h = jnp.pad(x, ((0, NP - N), (0, 0)))
    g = _dense_first(4, 16)(h, _chunk_w(params[0][1]))

    res_in = None
    for L, (cin, cout) in enumerate(LAYER_DIMS):
        w_self, w_neigh, gamma, beta = params[L]
        k_, wc = _chunks(cout)
        partials = _sc_scatter(k_, wc)(g, src_p, dst_p, zeros16)

        if L in RES_START:
            res_in = h
        gb = jnp.stack([gamma, beta])
        has_skip = L in SKIP_LAYERS
        next_cout = LAYER_DIMS[L + 1][1] if L + 1 < len(LAYER_DIMS) else None
        args = [h, partials, w_self, gb]
        if has_skip:
            args.append(res_in)
        if next_cout is not None:
            args.append(_chunk_w(params[L + 1][1]))
            h, g = _combine(cin, cout, has_skip, next_cout)(*args)
        else:
            (h,) = _combine(cin, cout, has_skip, None)(*args)

    return h[:N]


# speedup vs baseline: 6.7086x; 1.3990x over previous
import functools

import jax
import jax.numpy as jnp
from jax import lax
from jax.experimental import pallas as pl
from jax.experimental.pallas import tpu as pltpu
from jax.experimental.pallas import tpu_sc as plsc

N = 50000
E = 800000
NP = 50176
RB = 512
NB = NP // RB
NWORK = 32
BATCH = 128
EPW = 25088
NB_E = EPW // BATCH
EP = NWORK * EPW
RPT = NP // 16

LAYER_DIMS = [(4, 16),
              (16, 16), (16, 16), (16, 16), (16, 16),
              (16, 32), (32, 32), (32, 32), (32, 32), (32, 32),
              (32, 64), (64, 64), (64, 64), (64, 64), (64, 64),
              (64, 128), (128, 128), (128, 128), (128, 128), (128, 128)]
SKIP_LAYERS = frozenset({2, 4, 7, 9, 12, 14, 17, 19})
RES_START = frozenset({1, 3, 6, 8, 11, 13, 16, 18})


def _chunks(cout):
    wc = min(cout, 16)
    return cout // wc, wc


@functools.lru_cache(maxsize=None)
def _dense_first(cin, cout):
    k_, wc = _chunks(cout)

    def body(h_ref, w_ref, g_ref):
        h = h_ref[...]
        for c in range(k_):
            g_ref[c] = jnp.dot(h, w_ref[c], preferred_element_type=jnp.float32)

    return pl.pallas_call(
        body,
        grid=(NB,),
        in_specs=[
            pl.BlockSpec((RB, cin), lambda i: (i, 0)),
            pl.BlockSpec((k_, cin, wc), lambda i: (0, 0, 0)),
        ],
        out_specs=pl.BlockSpec((k_, RB, wc), lambda i: (0, i, 0)),
        out_shape=jax.ShapeDtypeStruct((k_, NP, wc), jnp.float32),
    )


@functools.lru_cache(maxsize=None)
def _combine(cin, cout, has_skip, next_cout):
    k_, wc = _chunks(cout)
    if next_cout is not None:
        kn, wcn = _chunks(next_cout)
    else:
        kn, wcn = 1, 0

    def body(*refs):
        it = iter(refs)
        h_ref = next(it)
        p_ref = next(it)
        ws_ref = next(it)
        gb_ref = next(it)
        skip_ref = next(it) if has_skip else None
        wn_ref = next(it) if next_cout is not None else None
        hout_ref = next(it)
        gout_ref = next(it) if next_cout is not None else None

        p = p_ref[...]
        neigh = jnp.concatenate([p[c, 0] + p[c, 1] for c in range(k_)], axis=-1)
        out = jnp.dot(h_ref[...], ws_ref[...], preferred_element_type=jnp.float32)
        out = (out + neigh) * gb_ref[0] + gb_ref[1]
        if skip_ref is not None:
            out = out + skip_ref[...]
        out = jnp.maximum(out, 0.0)
        rows = pl.program_id(0) * RB + lax.broadcasted_iota(jnp.int32, (RB, 1), 0)
        out = jnp.where(rows < N, out, 0.0)
        hout_ref[...] = out
        if gout_ref is not None:
            for c in range(kn):
                gout_ref[c] = jnp.dot(out, wn_ref[c],
                                      preferred_element_type=jnp.float32)

    in_specs = [
        pl.BlockSpec((RB, cin), lambda i: (i, 0)),
        pl.BlockSpec((k_, 2, RB, wc), lambda i: (0, 0, i, 0)),
        pl.BlockSpec((cin, cout), lambda i: (0, 0)),
        pl.BlockSpec((2, cout), lambda i: (0, 0)),
    ]
    if has_skip:
        in_specs.append(pl.BlockSpec((RB, cout), lambda i: (i, 0)))
    out_specs = [pl.BlockSpec((RB, cout), lambda i: (i, 0))]
    out_shape = [jax.ShapeDtypeStruct((NP, cout), jnp.float32)]
    if next_cout is not None:
        in_specs.append(pl.BlockSpec((kn, cout, wcn), lambda i: (0, 0, 0)))
        out_specs.append(pl.BlockSpec((kn, RB, wcn), lambda i: (0, i, 0)))
        out_shape.append(jax.ShapeDtypeStruct((kn, NP, wcn), jnp.float32))

    return pl.pallas_call(
        body,
        grid=(NB,),
        in_specs=in_specs,
        out_specs=out_specs,
        out_shape=out_shape,
    )


SUB = 2
BOP = SUB * BATCH
NOP = EPW // BOP
NBUF = 7
ROUNDS = NOP // NBUF
ZR = 64


@functools.lru_cache(maxsize=None)
def _sc_scatter(k_, wc):
    mesh = plsc.VectorSubcoreMesh(core_axis_name="c", subcore_axis_name="s")

    @functools.partial(
        pl.kernel,
        mesh=mesh,
        out_type=jax.ShapeDtypeStruct((k_, 2, NP, wc), jnp.float32),
        scratch_types=[
            pltpu.VMEM((EPW,), jnp.int32),
            pltpu.VMEM((EPW,), jnp.int32),
            pltpu.VMEM((NBUF, BOP, wc), jnp.float32),
            pltpu.VMEM((ZR, wc), jnp.float32),
            pltpu.VMEM_SHARED((NP, wc), jnp.float32),
            [pltpu.SemaphoreType.DMA] * NBUF,
            [pltpu.SemaphoreType.DMA] * NBUF,
            pltpu.SemaphoreType.DMA,
        ],
        compiler_params=pltpu.CompilerParams(use_tc_tiling_on_sc=False),
    )
    def sc_fn(g_hbm, src_hbm, dst_hbm, zer_hbm, out_hbm,
              src_v, dst_v, rows_v, zbuf_v, acc, sem_g, sem_s, sem_z):
        c = lax.axis_index("c")
        s = lax.axis_index("s")
        wid = c * 16 + s
        pltpu.sync_copy(src_hbm.at[wid], src_v)
        pltpu.sync_copy(dst_hbm.at[wid], dst_v)
        pltpu.sync_copy(zer_hbm, zbuf_v)
        base = s * RPT
        nz = RPT // ZR
        rem = RPT % ZR

        def zero_stripe():
            zd = []
            for i in range(nz):
                zd.append(pltpu.async_copy(
                    zbuf_v, acc.at[pl.ds(base + i * ZR, ZR)], sem_z))
            if rem:
                zd.append(pltpu.async_copy(
                    zbuf_v.at[pl.ds(0, rem)],
                    acc.at[pl.ds(base + RPT - rem, rem)], sem_z))
            for d in zd:
                d.wait()

        def gather(k, b, j):
            return pltpu.async_copy(
                g_hbm.at[k].at[src_v.at[pl.ds(j * BOP, BOP)]],
                rows_v.at[b], sem_g[b])

        def scat(b, j):
            return pltpu.async_copy(
                rows_v.at[b], acc.at[dst_v.at[pl.ds(j * BOP, BOP)]],
                sem_s[b], add=True)

        zero_stripe()
        for k in range(k_):
            plsc.subcore_barrier()

            for b in range(NBUF):
                gather(k, b, b)

            def round_body(jb, _):
                for b in range(NBUF):
                    pltpu.make_async_copy(
                        g_hbm.at[k].at[src_v.at[pl.ds((jb * NBUF + b) * BOP, BOP)]],
                        rows_v.at[b], sem_g[b]).wait()
                    scat(b, jb * NBUF + b)
                for b in range(NBUF):
                    pltpu.make_async_copy(
                        rows_v.at[b], acc.at[dst_v.at[pl.ds((jb * NBUF + b) * BOP, BOP)]],
                        sem_s[b]).wait()
                    gather(k, b, (jb + 1) * NBUF + b)
                return 0

            lax.fori_loop(0, ROUNDS - 1, round_body, 0)
            jl = (ROUNDS - 1) * NBUF
            tail = []
            for b in range(NBUF):
                pltpu.make_async_copy(
                    g_hbm.at[k].at[src_v.at[pl.ds((jl + b) * BOP, BOP)]],
                    rows_v.at[b], sem_g[b]).wait()
                tail.append(scat(b, jl + b))
            for d in tail:
                d.wait()

            plsc.subcore_barrier()
            pltpu.sync_copy(acc.at[pl.ds(base, RPT)],
                            out_hbm.at[k].at[c].at[pl.ds(base, RPT)])
            if k + 1 < k_:
                zero_stripe()

    return sc_fn


def _chunk_w(w):
    cin, cout = w.shape
    k_, wc = _chunks(cout)
    return w.reshape(cin, k_, wc).transpose(1, 0, 2)


def kernel(x, edge_index, params):
    src = edge_index[0]
    dst = edge_index[1]
    pad = jnp.full((EP - E,), N, dtype=jnp.int32)
    src_p = jnp.concatenate([src, pad]).reshape(NWORK, EPW)
    dst_p = jnp.concatenate([dst, pad]).reshape(NWORK, EPW)
    zeros16 = jnp.zeros((ZR, 16), jnp.float32)

    h = jnp.pad(x, ((0, NP - N), (0, 0)))
    g = _dense_first(4, 16)(h, _chunk_w(params[0][1]))

    res_in = None
    for L, (cin, cout) in enumerate(LAYER_DIMS):
        w_self, w_neigh, gamma, beta = params[L]
        k_, wc = _chunks(cout)
        partials = _sc_scatter(k_, wc)(g, src_p, dst_p, zeros16)

        if L in RES_START:
            res_in = h
        gb = jnp.stack([gamma, beta])
        has_skip = L in SKIP_LAYERS
        next_cout = LAYER_DIMS[L + 1][1] if L + 1 < len(LAYER_DIMS) else None
        args = [h, partials, w_self, gb]
        if has_skip:
            args.append(res_in)
        if next_cout is not None:
            args.append(_chunk_w(params[L + 1][1]))
            h, g = _combine(cin, cout, has_skip, next_cout)(*args)
        else:
            (h,) = _combine(cin, cout, has_skip, None)(*args)

    return h[:N]


# speedup vs baseline: 6.8076x; 1.0148x over previous
import functools

import jax
import jax.numpy as jnp
from jax import lax
from jax.experimental import pallas as pl
from jax.experimental.pallas import tpu as pltpu
from jax.experimental.pallas import tpu_sc as plsc

N = 50000
E = 800000
NP = 50176
RB = 512
NB = NP // RB
NWORK = 32
BATCH = 128
EPW = 25088
NB_E = EPW // BATCH
EP = NWORK * EPW
RPT = NP // 16

LAYER_DIMS = [(4, 16),
              (16, 16), (16, 16), (16, 16), (16, 16),
              (16, 32), (32, 32), (32, 32), (32, 32), (32, 32),
              (32, 64), (64, 64), (64, 64), (64, 64), (64, 64),
              (64, 128), (128, 128), (128, 128), (128, 128), (128, 128)]
SKIP_LAYERS = frozenset({2, 4, 7, 9, 12, 14, 17, 19})
RES_START = frozenset({1, 3, 6, 8, 11, 13, 16, 18})


def _chunks(cout):
    wc = min(cout, 16)
    return cout // wc, wc


@functools.lru_cache(maxsize=None)
def _dense_first(cin, cout):
    k_, wc = _chunks(cout)

    def body(h_ref, w_ref, g_ref):
        h = h_ref[...]
        for c in range(k_):
            g_ref[c] = jnp.dot(h, w_ref[c], preferred_element_type=jnp.float32)

    return pl.pallas_call(
        body,
        grid=(NB,),
        in_specs=[
            pl.BlockSpec((RB, cin), lambda i: (i, 0)),
            pl.BlockSpec((k_, cin, wc), lambda i: (0, 0, 0)),
        ],
        out_specs=pl.BlockSpec((k_, RB, wc), lambda i: (0, i, 0)),
        out_shape=jax.ShapeDtypeStruct((k_, NP, wc), jnp.float32),
    )


@functools.lru_cache(maxsize=None)
def _combine(cin, cout, has_skip, next_cout):
    k_, wc = _chunks(cout)
    if next_cout is not None:
        kn, wcn = _chunks(next_cout)
    else:
        kn, wcn = 1, 0

    def body(*refs):
        it = iter(refs)
        h_ref = next(it)
        p_ref = next(it)
        ws_ref = next(it)
        gb_ref = next(it)
        skip_ref = next(it) if has_skip else None
        wn_ref = next(it) if next_cout is not None else None
        hout_ref = next(it)
        gout_ref = next(it) if next_cout is not None else None

        p = p_ref[...]
        neigh = jnp.concatenate([p[c, 0] + p[c, 1] for c in range(k_)], axis=-1)
        out = jnp.dot(h_ref[...], ws_ref[...], preferred_element_type=jnp.float32)
        out = (out + neigh) * gb_ref[0] + gb_ref[1]
        if skip_ref is not None:
            out = out + skip_ref[...]
        out = jnp.maximum(out, 0.0)
        rows = pl.program_id(0) * RB + lax.broadcasted_iota(jnp.int32, (RB, 1), 0)
        out = jnp.where(rows < N, out, 0.0)
        hout_ref[...] = out
        if gout_ref is not None:
            for c in range(kn):
                gout_ref[c] = jnp.dot(out, wn_ref[c],
                                      preferred_element_type=jnp.float32)

    in_specs = [
        pl.BlockSpec((RB, cin), lambda i: (i, 0)),
        pl.BlockSpec((k_, 2, RB, wc), lambda i: (0, 0, i, 0)),
        pl.BlockSpec((cin, cout), lambda i: (0, 0)),
        pl.BlockSpec((2, cout), lambda i: (0, 0)),
    ]
    if has_skip:
        in_specs.append(pl.BlockSpec((RB, cout), lambda i: (i, 0)))
    out_specs = [pl.BlockSpec((RB, cout), lambda i: (i, 0))]
    out_shape = [jax.ShapeDtypeStruct((NP, cout), jnp.float32)]
    if next_cout is not None:
        in_specs.append(pl.BlockSpec((kn, cout, wcn), lambda i: (0, 0, 0)))
        out_specs.append(pl.BlockSpec((kn, RB, wcn), lambda i: (0, i, 0)))
        out_shape.append(jax.ShapeDtypeStruct((kn, NP, wcn), jnp.float32))

    return pl.pallas_call(
        body,
        grid=(NB,),
        in_specs=in_specs,
        out_specs=out_specs,
        out_shape=out_shape,
    )


SUB = 2
BOP = SUB * BATCH
NOP = EPW // BOP
NBUF = 7
ROUNDS = NOP // NBUF
ZR = 64


@functools.lru_cache(maxsize=None)
def _sc_scatter(k_, wc):
    mesh = plsc.VectorSubcoreMesh(core_axis_name="c", subcore_axis_name="s")

    @functools.partial(
        pl.kernel,
        mesh=mesh,
        out_type=jax.ShapeDtypeStruct((k_, 2, NP, wc), jnp.float32),
        scratch_types=[
            pltpu.VMEM((EPW,), jnp.int32),
            pltpu.VMEM((EPW,), jnp.int32),
            pltpu.VMEM((NBUF, BOP, wc), jnp.float32),
            pltpu.VMEM((ZR, wc), jnp.float32),
            pltpu.VMEM_SHARED((NP, wc), jnp.float32),
            [pltpu.SemaphoreType.DMA] * NBUF,
            [pltpu.SemaphoreType.DMA] * NBUF,
            pltpu.SemaphoreType.DMA,
        ],
        compiler_params=pltpu.CompilerParams(use_tc_tiling_on_sc=False),
    )
    def sc_fn(g_hbm, src_hbm, dst_hbm, zer_hbm, out_hbm,
              src_v, dst_v, rows_v, zbuf_v, acc, sem_g, sem_s, sem_z):
        c = lax.axis_index("c")
        s = lax.axis_index("s")
        wid = c * 16 + s
        d_src = pltpu.async_copy(src_hbm.at[wid], src_v, sem_g[0])
        d_dst = pltpu.async_copy(dst_hbm.at[wid], dst_v, sem_g[1])
        d_z = pltpu.async_copy(zer_hbm, zbuf_v, sem_z)
        base = s * RPT
        nz = RPT // ZR
        rem = RPT % ZR

        def zero_stripe():
            zd = []
            for i in range(nz):
                zd.append(pltpu.async_copy(
                    zbuf_v, acc.at[pl.ds(base + i * ZR, ZR)], sem_z))
            if rem:
                zd.append(pltpu.async_copy(
                    zbuf_v.at[pl.ds(0, rem)],
                    acc.at[pl.ds(base + RPT - rem, rem)], sem_z))
            for d in zd:
                d.wait()

        def gather(k, b, j):
            return pltpu.async_copy(
                g_hbm.at[k].at[src_v.at[pl.ds(j * BOP, BOP)]],
                rows_v.at[b], sem_g[b])

        def scat(b, j):
            return pltpu.async_copy(
                rows_v.at[b], acc.at[dst_v.at[pl.ds(j * BOP, BOP)]],
                sem_s[b], add=True)

        d_z.wait()
        zero_stripe()
        d_src.wait()
        d_dst.wait()
        for b in range(NBUF):
            gather(0, b, b)

        for k in range(k_):
            plsc.subcore_barrier()

            def round_body(jb, _):
                for b in range(NBUF):
                    pltpu.make_async_copy(
                        g_hbm.at[k].at[src_v.at[pl.ds((jb * NBUF + b) * BOP, BOP)]],
                        rows_v.at[b], sem_g[b]).wait()
                    scat(b, jb * NBUF + b)
                for b in range(NBUF):
                    pltpu.make_async_copy(
                        rows_v.at[b], acc.at[dst_v.at[pl.ds((jb * NBUF + b) * BOP, BOP)]],
                        sem_s[b]).wait()
                    gather(k, b, (jb + 1) * NBUF + b)
                return 0

            lax.fori_loop(0, ROUNDS - 1, round_body, 0)
            jl = (ROUNDS - 1) * NBUF
            tail = []
            for b in range(NBUF):
                pltpu.make_async_copy(
                    g_hbm.at[k].at[src_v.at[pl.ds((jl + b) * BOP, BOP)]],
                    rows_v.at[b], sem_g[b]).wait()
                tail.append(scat(b, jl + b))
            for d in tail:
                d.wait()

            plsc.subcore_barrier()
            dr = pltpu.async_copy(acc.at[pl.ds(base, RPT)],
                                  out_hbm.at[k].at[c].at[pl.ds(base, RPT)],
                                  sem_z)
            if k + 1 < k_:
                for b in range(NBUF):
                    gather(k + 1, b, b)
                dr.wait()
                zero_stripe()
            else:
                dr.wait()

    return sc_fn


def _chunk_w(w):
    cin, cout = w.shape
    k_, wc = _chunks(cout)
    return w.reshape(cin, k_, wc).transpose(1, 0, 2)


def kernel(x, edge_index, params):
    src = edge_index[0]
    dst = edge_index[1]
    pad = jnp.full((EP - E,), N, dtype=jnp.int32)
    src_p = jnp.concatenate([src, pad]).reshape(NWORK, EPW)
    dst_p = jnp.concatenate([dst, pad]).reshape(NWORK, EPW)
    zeros16 = jnp.zeros((ZR, 16), jnp.float32)

    h = jnp.pad(x, ((0, NP - N), (0, 0)))
    g = _dense_first(4, 16)(h, _chunk_w(params[0][1]))

    res_in = None
    for L, (cin, cout) in enumerate(LAYER_DIMS):
        w_self, w_neigh, gamma, beta = params[L]
        k_, wc = _chunks(cout)
        partials = _sc_scatter(k_, wc)(g, src_p, dst_p, zeros16)

        if L in RES_START:
            res_in = h
        gb = jnp.stack([gamma, beta])
        has_skip = L in SKIP_LAYERS
        next_cout = LAYER_DIMS[L + 1][1] if L + 1 < len(LAYER_DIMS) else None
        args = [h, partials, w_self, gb]
        if has_skip:
            args.append(res_in)
        if next_cout is not None:
            args.append(_chunk_w(params[L + 1][1]))
            h, g = _combine(cin, cout, has_skip, next_cout)(*args)
        else:
            (h,) = _combine(cin, cout, has_skip, None)(*args)

    return h[:N]
